# baseline (speedup 1.0000x reference)
import functools

import jax
import jax.numpy as jnp
from jax import lax
from jax.experimental import pallas as pl
from jax.experimental.pallas import tpu as pltpu
from jax.experimental.pallas import tpu_sc as plsc

NUM_LEVELS_ = 65536
ROWS = 16384
COLS = 200
NW = 32
ROWS_PER_W = ROWS // NW
RCHUNK = 32
N_CHUNKS = ROWS_PER_W // RCHUNK
N_PAIRS = N_CHUNKS // 2
LANES = 16
N_FULL = COLS // LANES
TAIL = COLS - LANES


def _sc_body(x_hbm, noise_hbm, gs_hbm, out_hbm, table,
             xb0, xb1, nb0, nb1, ob0, ob1,
             sem_t, sem_x0, sem_x1, sem_n0, sem_n1, sem_o0, sem_o1):
    xb = (xb0, xb1)
    nb = (nb0, nb1)
    ob = (ob0, ob1)
    wid = lax.axis_index("s") * 2 + lax.axis_index("c")
    base = wid * ROWS_PER_W
    sx = (sem_x0, sem_x1)
    sn = (sem_n0, sem_n1)
    so = (sem_o0, sem_o1)

    cp_t = pltpu.async_copy(gs_hbm, table, sem_t)

    def start_in(k, b):
        off = base + k * RCHUNK
        pltpu.async_copy(x_hbm.at[pl.ds(off, RCHUNK)], xb[b], sx[b])
        pltpu.async_copy(noise_hbm.at[pl.ds(off, RCHUNK)], nb[b], sn[b])

    start_in(0, 0)
    start_in(1, 1)

    cp_t.wait()
    smin = table[pl.ds(0, LANES)][0]
    smax = table[pl.ds(NUM_LEVELS_ - LANES, LANES)][LANES - 1]
    inv_range = 1.0 / jnp.full((LANES,), smax - smin, jnp.float32)
    c0 = -smin * inv_range

    def pair_body(g, _):
        for b in range(2):
            k = 2 * g + b
            off = base + k * RCHUNK
            xr, nr, orr = xb[b], nb[b], ob[b]
            pltpu.make_async_copy(x_hbm.at[pl.ds(off, RCHUNK)], xr, sx[b]).wait()
            pltpu.make_async_copy(noise_hbm.at[pl.ds(off, RCHUNK)], nr, sn[b]).wait()

            @pl.when(g > 0)
            def _():
                pltpu.make_async_copy(orr, out_hbm.at[pl.ds(off, RCHUNK)], so[b]).wait()

            @plsc.parallel_loop(0, RCHUNK, step=1, unroll=16)
            def _inner(r):
                for j in list(range(N_FULL)) + [-1]:
                    s = TAIL if j < 0 else j * LANES
                    xs = xr[r, pl.ds(s, LANES)]
                    ns = nr[r, pl.ds(s, LANES)]
                    t = xs * float(NUM_LEVELS_ - 1)
                    ii = t.astype(jnp.int32)
                    ic = jnp.minimum(ii + 1, NUM_LEVELS_ - 1)
                    alpha = t - ii.astype(jnp.float32)
                    vf = plsc.load_gather(table, [ii])
                    vc = plsc.load_gather(table, [ic])
                    sv = vf + alpha * (vc - vf)
                    orr[r, pl.ds(s, LANES)] = sv * inv_range + ns * 0.03 + c0

            pltpu.async_copy(orr, out_hbm.at[pl.ds(off, RCHUNK)], so[b])

            @pl.when(g < N_PAIRS - 1)
            def _():
                start_in(k + 2, b)
        return 0

    lax.fori_loop(0, N_PAIRS, pair_body, 0)
    for b in range(2):
        pltpu.make_async_copy(ob[b], out_hbm.at[pl.ds(base, RCHUNK)], so[b]).wait()


@jax.jit
def _sc_call(x, noise, gs):
    mesh = plsc.VectorSubcoreMesh(core_axis_name="c", subcore_axis_name="s")
    return pl.kernel(
        _sc_body,
        out_type=jax.ShapeDtypeStruct((ROWS, COLS), jnp.float32),
        mesh=mesh,
        compiler_params=pltpu.CompilerParams(
            needs_layout_passes=False, use_tc_tiling_on_sc=True),
        scratch_types=[
            pltpu.VMEM((NUM_LEVELS_,), jnp.float32),
            pltpu.VMEM((RCHUNK, COLS), jnp.float32),
            pltpu.VMEM((RCHUNK, COLS), jnp.float32),
            pltpu.VMEM((RCHUNK, COLS), jnp.float32),
            pltpu.VMEM((RCHUNK, COLS), jnp.float32),
            pltpu.VMEM((RCHUNK, COLS), jnp.float32),
            pltpu.VMEM((RCHUNK, COLS), jnp.float32),
            pltpu.SemaphoreType.DMA,
            pltpu.SemaphoreType.DMA,
            pltpu.SemaphoreType.DMA,
            pltpu.SemaphoreType.DMA,
            pltpu.SemaphoreType.DMA,
            pltpu.SemaphoreType.DMA,
            pltpu.SemaphoreType.DMA,
        ],
    )(x, noise, gs)


def kernel(x, good_sensor, noise):
    return _sc_call(x, noise, good_sensor)


# speedup vs baseline: 1.3940x; 1.3940x over previous
import functools

import jax
import jax.numpy as jnp
from jax import lax
from jax.experimental import pallas as pl
from jax.experimental.pallas import tpu as pltpu
from jax.experimental.pallas import tpu_sc as plsc

NUM_LEVELS_ = 65536
ROWS = 16384
COLS = 200
NW = 32
ROWS_PER_W = ROWS // NW
RCHUNK = 32
N_CHUNKS = ROWS_PER_W // RCHUNK
N_PAIRS = N_CHUNKS // 2
LANES = 16
N_FULL = COLS // LANES
TAIL = COLS - LANES


def _sc_body(x_hbm, noise_hbm, gs_hbm, out_hbm, table,
             xb0, xb1, nb0, nb1, ob0, ob1,
             sem_t, sem_x0, sem_x1, sem_n0, sem_n1, sem_o0, sem_o1):
    xb = (xb0, xb1)
    nb = (nb0, nb1)
    ob = (ob0, ob1)
    wid = lax.axis_index("s") * 2 + lax.axis_index("c")
    base = wid * ROWS_PER_W
    sx = (sem_x0, sem_x1)
    sn = (sem_n0, sem_n1)
    so = (sem_o0, sem_o1)

    cp_t = pltpu.async_copy(gs_hbm, table, sem_t)

    def start_in(k, b):
        off = base + k * RCHUNK
        pltpu.async_copy(x_hbm.at[pl.ds(off, RCHUNK)], xb[b], sx[b])
        pltpu.async_copy(noise_hbm.at[pl.ds(off, RCHUNK)], nb[b], sn[b])

    start_in(0, 0)
    start_in(1, 1)

    cp_t.wait()
    smin = table[pl.ds(0, LANES)][0]
    smax = table[pl.ds(NUM_LEVELS_ - LANES, LANES)][LANES - 1]
    inv_range = 1.0 / jnp.full((LANES,), smax - smin, jnp.float32)
    c0 = -smin * inv_range

    def pair_body(g, _):
        for b in range(2):
            k = 2 * g + b
            off = base + k * RCHUNK
            xr, nr, orr = xb[b], nb[b], ob[b]
            pltpu.make_async_copy(x_hbm.at[pl.ds(off, RCHUNK)], xr, sx[b]).wait()
            pltpu.make_async_copy(noise_hbm.at[pl.ds(off, RCHUNK)], nr, sn[b]).wait()

            @pl.when(g > 0)
            def _():
                pltpu.make_async_copy(orr, out_hbm.at[pl.ds(off, RCHUNK)], so[b]).wait()

            @plsc.parallel_loop(0, RCHUNK, step=1, unroll=8)
            def _inner(r):
                for j in list(range(N_FULL)) + [-1]:
                    s = TAIL if j < 0 else j * LANES
                    xs = xr[r, pl.ds(s, LANES)]
                    ns = nr[r, pl.ds(s, LANES)]
                    t = xs * float(NUM_LEVELS_ - 1)
                    ii = t.astype(jnp.int32)
                    ic = jnp.minimum(ii + 1, NUM_LEVELS_ - 1)
                    alpha = t - ii.astype(jnp.float32)
                    vf = plsc.load_gather(table, [ii])
                    vc = plsc.load_gather(table, [ic])
                    sv = vf + alpha * (vc - vf)
                    orr[r, pl.ds(s, LANES)] = sv * inv_range + ns * 0.03 + c0

            pltpu.async_copy(orr, out_hbm.at[pl.ds(off, RCHUNK)], so[b])

            @pl.when(g < N_PAIRS - 1)
            def _():
                start_in(k + 2, b)
        return 0

    lax.fori_loop(0, N_PAIRS, pair_body, 0)
    for b in range(2):
        pltpu.make_async_copy(ob[b], out_hbm.at[pl.ds(base, RCHUNK)], so[b]).wait()


@jax.jit
def _sc_call(x, noise, gs):
    mesh = plsc.VectorSubcoreMesh(core_axis_name="c", subcore_axis_name="s")
    return pl.kernel(
        _sc_body,
        out_type=jax.ShapeDtypeStruct((ROWS, COLS), jnp.float32),
        mesh=mesh,
        compiler_params=pltpu.CompilerParams(
            needs_layout_passes=False, use_tc_tiling_on_sc=True),
        scratch_types=[
            pltpu.VMEM((NUM_LEVELS_,), jnp.float32),
            pltpu.VMEM((RCHUNK, COLS), jnp.float32),
            pltpu.VMEM((RCHUNK, COLS), jnp.float32),
            pltpu.VMEM((RCHUNK, COLS), jnp.float32),
            pltpu.VMEM((RCHUNK, COLS), jnp.float32),
            pltpu.VMEM((RCHUNK, COLS), jnp.float32),
            pltpu.VMEM((RCHUNK, COLS), jnp.float32),
            pltpu.SemaphoreType.DMA,
            pltpu.SemaphoreType.DMA,
            pltpu.SemaphoreType.DMA,
            pltpu.SemaphoreType.DMA,
            pltpu.SemaphoreType.DMA,
            pltpu.SemaphoreType.DMA,
            pltpu.SemaphoreType.DMA,
        ],
    )(x, noise, gs)


def kernel(x, good_sensor, noise):
    return _sc_call(x, noise, good_sensor)


# speedup vs baseline: 2.2736x; 1.6310x over previous
import functools

import jax
import jax.numpy as jnp
from jax import lax
from jax.experimental import pallas as pl
from jax.experimental.pallas import tpu as pltpu
from jax.experimental.pallas import tpu_sc as plsc

NUM_LEVELS_ = 65536
ROWS_T = 200
COLS_T = 16384
NW = 32
UNIT = 2048
UNITS_PER_ROW = COLS_T // UNIT
N_UNITS = ROWS_T * UNITS_PER_ROW
PER_W = N_UNITS // NW
N_PAIRS = PER_W // 2
LANES = 16


def _sc_body(x_hbm, noise_hbm, gs_hbm, out_hbm, table,
             xb0, xb1, nb0, nb1, ob0, ob1,
             sem_t, sem_x0, sem_x1, sem_n0, sem_n1, sem_o0, sem_o1):
    xb = (xb0, xb1)
    nb = (nb0, nb1)
    ob = (ob0, ob1)
    wid = lax.axis_index("s") * 2 + lax.axis_index("c")
    base = wid * PER_W
    sx = (sem_x0, sem_x1)
    sn = (sem_n0, sem_n1)
    so = (sem_o0, sem_o1)

    def slab(k):
        uid = base + k
        return uid // UNITS_PER_ROW, (uid % UNITS_PER_ROW) * UNIT

    cp_t = pltpu.async_copy(gs_hbm, table, sem_t)

    def start_in(k, b):
        r, c = slab(k)
        pltpu.async_copy(x_hbm.at[r, pl.ds(c, UNIT)], xb[b], sx[b])
        pltpu.async_copy(noise_hbm.at[r, pl.ds(c, UNIT)], nb[b], sn[b])

    start_in(0, 0)
    start_in(1, 1)

    cp_t.wait()
    smin = table[pl.ds(0, LANES)][0]
    smax = table[pl.ds(NUM_LEVELS_ - LANES, LANES)][LANES - 1]
    inv_range = 1.0 / jnp.full((LANES,), smax - smin, jnp.float32)
    c0 = -smin * inv_range

    def pair_body(g, _):
        for b in range(2):
            k = 2 * g + b
            r, c = slab(k)
            xr, nr, orr = xb[b], nb[b], ob[b]
            pltpu.make_async_copy(x_hbm.at[r, pl.ds(c, UNIT)], xr, sx[b]).wait()
            pltpu.make_async_copy(noise_hbm.at[r, pl.ds(c, UNIT)], nr, sn[b]).wait()

            @pl.when(g > 0)
            def _():
                pltpu.make_async_copy(orr, out_hbm.at[r, pl.ds(c, UNIT)], so[b]).wait()

            @plsc.parallel_loop(0, UNIT, step=LANES, unroll=16)
            def _inner(s):
                xs = xr[pl.ds(s, LANES)]
                ns = nr[pl.ds(s, LANES)]
                t = xs * float(NUM_LEVELS_ - 1)
                ii = t.astype(jnp.int32)
                ic = jnp.minimum(ii + 1, NUM_LEVELS_ - 1)
                alpha = t - ii.astype(jnp.float32)
                vf = plsc.load_gather(table, [ii])
                vc = plsc.load_gather(table, [ic])
                sv = vf + alpha * (vc - vf)
                orr[pl.ds(s, LANES)] = sv * inv_range + ns * 0.03 + c0

            pltpu.async_copy(orr, out_hbm.at[r, pl.ds(c, UNIT)], so[b])

            @pl.when(g < N_PAIRS - 1)
            def _():
                start_in(k + 2, b)
        return 0

    lax.fori_loop(0, N_PAIRS, pair_body, 0)
    r0, c0_ = slab(0)
    for b in range(2):
        pltpu.make_async_copy(ob[b], out_hbm.at[r0, pl.ds(c0_, UNIT)], so[b]).wait()


@jax.jit
def _sc_call(xt, noiset, gs):
    mesh = plsc.VectorSubcoreMesh(core_axis_name="c", subcore_axis_name="s")
    return pl.kernel(
        _sc_body,
        out_type=jax.ShapeDtypeStruct((ROWS_T, COLS_T), jnp.float32),
        mesh=mesh,
        compiler_params=pltpu.CompilerParams(
            needs_layout_passes=False, use_tc_tiling_on_sc=True),
        scratch_types=[
            pltpu.VMEM((NUM_LEVELS_,), jnp.float32),
            pltpu.VMEM((UNIT,), jnp.float32),
            pltpu.VMEM((UNIT,), jnp.float32),
            pltpu.VMEM((UNIT,), jnp.float32),
            pltpu.VMEM((UNIT,), jnp.float32),
            pltpu.VMEM((UNIT,), jnp.float32),
            pltpu.VMEM((UNIT,), jnp.float32),
            pltpu.SemaphoreType.DMA,
            pltpu.SemaphoreType.DMA,
            pltpu.SemaphoreType.DMA,
            pltpu.SemaphoreType.DMA,
            pltpu.SemaphoreType.DMA,
            pltpu.SemaphoreType.DMA,
            pltpu.SemaphoreType.DMA,
        ],
    )(xt, noiset, gs)


def kernel(x, good_sensor, noise):
    return _sc_call(x.T, noise.T, good_sensor).T
